# baseline (speedup 1.0000x reference)
import functools

import jax
import jax.numpy as jnp
from jax.experimental import pallas as pl
from jax.experimental.pallas import tpu as pltpu
from jax.sharding import Mesh, PartitionSpec as P

S = 2048
DIM = 2048
H = 16
DH = 128
BLOCK = 128
NB = S // BLOCK
WIN = 2
EPS = 1e-6
SCALE = 1.0 / (DH ** 0.5)

STRIP = 256
WBLK = STRIP // BLOCK + 2 * WIN
WK = WBLK * BLOCK

NT = 4
TILE_N = DIM // NT

NDEV = 1
try:
    NDEV = 2 if len(jax.devices()) >= 2 else 1
except Exception:
    NDEV = 1

LS = S // NDEV
HALO = WIN * BLOCK
EXT = LS + 2 * HALO
NSTRIP_L = LS // STRIP


def _qkv_kernel(x_ref, wq_ref, wk_ref, wv_ref, gq_ref, gk_ref, o_ref,
                xs_ref):
    n = pl.program_id(0)

    @pl.when(n == 0)
    def _():
        xs_ref[...] = x_ref[...].astype(jnp.bfloat16)

    def _mm(w_ref):
        w = w_ref[...].astype(jnp.bfloat16)
        return jnp.dot(xs_ref[...], w, preferred_element_type=jnp.float32)

    def _norm(acc, g):
        segs = []
        for j in range(TILE_N // DH):
            seg = acc[:, j * DH:(j + 1) * DH]
            var = jnp.mean(seg * seg, axis=1, keepdims=True)
            segs.append(seg * jax.lax.rsqrt(var + EPS))
        gfull = jnp.concatenate([g] * (TILE_N // DH))
        return (jnp.concatenate(segs, axis=1) * gfull[None, :]).astype(jnp.bfloat16)

    @pl.when(n < NT)
    def _():
        o_ref[...] = _norm(_mm(wq_ref), gq_ref[...])

    @pl.when((n >= NT) & (n < 2 * NT))
    def _():
        o_ref[...] = _norm(_mm(wk_ref), gk_ref[...])

    @pl.when(n >= 2 * NT)
    def _():
        o_ref[...] = _mm(wv_ref).astype(jnp.bfloat16)


def _attn_kernel(info_ref, q_ref, k_ref, v_ref, wo_ref, o_ref):
    s = pl.program_id(0)
    ext_start = info_ref[s, 0] * BLOCK
    qb0 = info_ref[s, 1]
    sbg = info_ref[s, 2]

    r = jax.lax.broadcasted_iota(jnp.int32, (STRIP, WK), 0)
    c = jax.lax.broadcasted_iota(jnp.int32, (STRIP, WK), 1)
    qb = qb0 + r // BLOCK
    jb = sbg + c // BLOCK
    neg = jnp.where(jnp.abs(jb - qb) <= WIN,
                    jnp.float32(0), jnp.float32(-1e9))

    outs = []
    for h in range(H):
        lo, hi = h * DH, (h + 1) * DH
        qh = q_ref[:, lo:hi]
        kh = k_ref[pl.ds(ext_start, WK), lo:hi]
        vh = v_ref[pl.ds(ext_start, WK), lo:hi]
        sc = jax.lax.dot_general(
            qh, kh, (((1,), (1,)), ((), ())),
            preferred_element_type=jnp.float32) * SCALE + neg
        m = jnp.max(sc, axis=1, keepdims=True)
        p = jnp.exp(sc - m)
        l = jnp.sum(p, axis=1, keepdims=True)
        oh = jnp.dot(p.astype(jnp.bfloat16), vh,
                     preferred_element_type=jnp.float32)
        outs.append((oh / l).astype(jnp.bfloat16))

    a = jnp.concatenate(outs, axis=1)
    o_ref[...] = jnp.dot(a, wo_ref[...], preferred_element_type=jnp.float32)


def _local(x, Wq, Wk, Wv, Wo, gq, gk):
    d = jax.lax.axis_index('seq').astype(jnp.int32)

    qkv = pl.pallas_call(
        _qkv_kernel,
        grid=(3 * NT,),
        in_specs=[
            pl.BlockSpec((LS, DIM), lambda n: (0, 0)),
            pl.BlockSpec((DIM, TILE_N),
                         lambda n: (0, jnp.clip(n, 0, NT - 1))),
            pl.BlockSpec((DIM, TILE_N),
                         lambda n: (0, jnp.clip(n - NT, 0, NT - 1))),
            pl.BlockSpec((DIM, TILE_N),
                         lambda n: (0, jnp.clip(n - 2 * NT, 0, NT - 1))),
            pl.BlockSpec((DH,), lambda n: (0,)),
            pl.BlockSpec((DH,), lambda n: (0,)),
        ],
        out_specs=pl.BlockSpec((LS, TILE_N), lambda n: (0, n)),
        out_shape=jax.ShapeDtypeStruct((LS, 3 * DIM), jnp.bfloat16),
        scratch_shapes=[pltpu.VMEM((LS, DIM), jnp.bfloat16)],
    )(x, Wq, Wk, Wv, gq, gk)

    kv = qkv[:, DIM:]
    fwd = [(i, i + 1) for i in range(NDEV - 1)]
    bwd = [(i + 1, i) for i in range(NDEV - 1)]
    halo_lo = jax.lax.ppermute(kv[LS - HALO:, :], 'seq', fwd)
    halo_hi = jax.lax.ppermute(kv[:HALO, :], 'seq', bwd)
    k_ext = jnp.concatenate(
        [halo_lo[:, :DIM], qkv[:, DIM:2 * DIM], halo_hi[:, :DIM]], axis=0)
    v_ext = jnp.concatenate(
        [halo_lo[:, DIM:], qkv[:, 2 * DIM:], halo_hi[:, DIM:]], axis=0)

    sarr = jnp.arange(NSTRIP_L, dtype=jnp.int32)
    qb0 = d * (LS // BLOCK) + sarr * (STRIP // BLOCK)
    sbg = jnp.clip(qb0 - WIN, 0, NB - WBLK)
    ext_start_blk = sbg - (d * LS - HALO) // BLOCK
    info = jnp.stack([ext_start_blk, qb0, sbg], axis=1)

    out = pl.pallas_call(
        _attn_kernel,
        grid_spec=pltpu.PrefetchScalarGridSpec(
            num_scalar_prefetch=1,
            grid=(NSTRIP_L,),
            in_specs=[
                pl.BlockSpec((STRIP, DIM), lambda s, info: (s, 0)),
                pl.BlockSpec((EXT, DIM), lambda s, info: (0, 0)),
                pl.BlockSpec((EXT, DIM), lambda s, info: (0, 0)),
                pl.BlockSpec((DIM, DIM), lambda s, info: (0, 0)),
            ],
            out_specs=pl.BlockSpec((STRIP, DIM), lambda s, info: (s, 0)),
        ),
        out_shape=jax.ShapeDtypeStruct((LS, DIM), jnp.float32),
    )(info, qkv[:, :DIM], k_ext, v_ext, Wo.astype(jnp.bfloat16))

    return out


@jax.jit
def _run(x, Wq, Wk, Wv, Wo, gq, gk):
    mesh = Mesh(jax.devices()[:NDEV], ('seq',))
    return jax.shard_map(
        _local,
        mesh=mesh,
        in_specs=(P('seq', None), P(None, None), P(None, None),
                  P(None, None), P(None, None), P(None), P(None)),
        out_specs=P('seq', None),
        check_vma=False,
    )(x, Wq, Wk, Wv, Wo, gq, gk)


def kernel(x, Wq, Wk, Wv, Wo, gq, gk):
    return _run(x[0], Wq, Wk, Wv, Wo, gq, gk)[None]


# speedup vs baseline: 1.7022x; 1.7022x over previous
import jax
import jax.numpy as jnp
from jax.experimental import pallas as pl
from jax.experimental.pallas import tpu as pltpu
from jax.sharding import Mesh, PartitionSpec as P

S = 2048
DIM = 2048
H = 16
DH = 128
BLOCK = 128
NB = S // BLOCK
WIN = 2
EPS = 1e-6
SCALE = 1.0 / (DH ** 0.5)

STRIP = 256
WBLK = STRIP // BLOCK + 2 * WIN
WK = WBLK * BLOCK

NT = 4
TILE_N = DIM // NT

NDEV = 1
try:
    NDEV = 2 if len(jax.devices()) >= 2 else 1
except Exception:
    NDEV = 1

LS = S // NDEV
LSB = LS // BLOCK
EXTB = min(NB, LSB + 2 * WIN)
EXT = EXTB * BLOCK
NSTRIP_L = LS // STRIP
_E0 = [min(max(d * LSB - WIN, 0), NB - EXTB) for d in range(NDEV)]


def _qkv_kernel(info_ref, x_ref, wq_ref, wk_ref, wv_ref, gq_ref, gk_ref,
                q_ref, kv_ref, xs_ref):
    n = pl.program_id(0)
    qoff = info_ref[0] * BLOCK

    @pl.when(n == 0)
    def _():
        xs_ref[...] = x_ref[...].astype(jnp.bfloat16)

    def _mm(w_ref, rows):
        w = w_ref[...].astype(jnp.bfloat16)
        return jnp.dot(rows, w, preferred_element_type=jnp.float32)

    def _norm(acc, g):
        segs = []
        for j in range(TILE_N // DH):
            seg = acc[:, j * DH:(j + 1) * DH]
            var = jnp.mean(seg * seg, axis=1, keepdims=True)
            segs.append(seg * jax.lax.rsqrt(var + EPS))
        gfull = jnp.concatenate([g] * (TILE_N // DH))
        return (jnp.concatenate(segs, axis=1) * gfull[None, :]).astype(jnp.bfloat16)

    @pl.when(n < NT)
    def _():
        q_ref[...] = _norm(_mm(wq_ref, xs_ref[pl.ds(qoff, LS), :]),
                           gq_ref[...])

    @pl.when((n >= NT) & (n < 2 * NT))
    def _():
        kv_ref[...] = _norm(_mm(wk_ref, xs_ref[...]), gk_ref[...])

    @pl.when(n >= 2 * NT)
    def _():
        kv_ref[...] = _mm(wv_ref, xs_ref[...]).astype(jnp.bfloat16)


def _attn_kernel(info_ref, q_ref, k_ref, v_ref, wo_ref, o_ref):
    s = pl.program_id(0)
    ext_start = info_ref[s, 0] * BLOCK
    qb0 = info_ref[s, 1]
    sbg = info_ref[s, 2]

    r = jax.lax.broadcasted_iota(jnp.int32, (STRIP, WK), 0)
    c = jax.lax.broadcasted_iota(jnp.int32, (STRIP, WK), 1)
    qb = qb0 + r // BLOCK
    jb = sbg + c // BLOCK
    neg = jnp.where(jnp.abs(jb - qb) <= WIN,
                    jnp.float32(0), jnp.float32(-1e9))

    outs = []
    for h in range(H):
        lo, hi = h * DH, (h + 1) * DH
        qh = q_ref[:, lo:hi]
        kh = k_ref[pl.ds(ext_start, WK), lo:hi]
        vh = v_ref[pl.ds(ext_start, WK), lo:hi]
        sc = jax.lax.dot_general(
            qh, kh, (((1,), (1,)), ((), ())),
            preferred_element_type=jnp.float32) * SCALE + neg
        m = jnp.max(sc, axis=1, keepdims=True)
        p = jnp.exp(sc - m)
        l = jnp.sum(p, axis=1, keepdims=True)
        oh = jnp.dot(p.astype(jnp.bfloat16), vh,
                     preferred_element_type=jnp.float32)
        outs.append((oh / l).astype(jnp.bfloat16))

    a = jnp.concatenate(outs, axis=1)
    o_ref[...] = jnp.dot(a, wo_ref[...], preferred_element_type=jnp.float32)


def _local(x_ext, Wq, Wk, Wv, Wo_bf, gq, gk):
    x_ext = x_ext.reshape(EXT, DIM)
    d = jax.lax.axis_index('seq').astype(jnp.int32)
    e0 = jnp.clip(d * LSB - WIN, 0, NB - EXTB)
    qoffb = d * LSB - e0

    q, kv = pl.pallas_call(
        _qkv_kernel,
        grid_spec=pltpu.PrefetchScalarGridSpec(
            num_scalar_prefetch=1,
            grid=(3 * NT,),
            in_specs=[
                pl.BlockSpec((EXT, DIM), lambda n, info: (0, 0)),
                pl.BlockSpec((DIM, TILE_N),
                             lambda n, info: (0, jnp.clip(n, 0, NT - 1))),
                pl.BlockSpec((DIM, TILE_N),
                             lambda n, info: (0, jnp.clip(n - NT, 0, NT - 1))),
                pl.BlockSpec((DIM, TILE_N),
                             lambda n, info: (0, jnp.clip(n - 2 * NT, 0, NT - 1))),
                pl.BlockSpec((DH,), lambda n, info: (0,)),
                pl.BlockSpec((DH,), lambda n, info: (0,)),
            ],
            out_specs=[
                pl.BlockSpec((LS, TILE_N),
                             lambda n, info: (0, jnp.clip(n, 0, NT - 1))),
                pl.BlockSpec((EXT, TILE_N),
                             lambda n, info: (0, jnp.clip(n - NT, 0, 2 * NT - 1))),
            ],
            scratch_shapes=[pltpu.VMEM((EXT, DIM), jnp.bfloat16)],
        ),
        out_shape=[
            jax.ShapeDtypeStruct((LS, DIM), jnp.bfloat16),
            jax.ShapeDtypeStruct((EXT, 2 * DIM), jnp.bfloat16),
        ],
    )(qoffb[None], x_ext, Wq, Wk, Wv, gq, gk)

    sarr = jnp.arange(NSTRIP_L, dtype=jnp.int32)
    qb0 = d * LSB + sarr * (STRIP // BLOCK)
    sbg = jnp.clip(qb0 - WIN, 0, NB - WBLK)
    info = jnp.stack([sbg - e0, qb0, sbg], axis=1)

    out = pl.pallas_call(
        _attn_kernel,
        grid_spec=pltpu.PrefetchScalarGridSpec(
            num_scalar_prefetch=1,
            grid=(NSTRIP_L,),
            in_specs=[
                pl.BlockSpec((STRIP, DIM), lambda s, info: (s, 0)),
                pl.BlockSpec((EXT, DIM), lambda s, info: (0, 0)),
                pl.BlockSpec((EXT, DIM), lambda s, info: (0, 1)),
                pl.BlockSpec((DIM, DIM), lambda s, info: (0, 0)),
            ],
            out_specs=pl.BlockSpec((STRIP, DIM), lambda s, info: (s, 0)),
        ),
        out_shape=jax.ShapeDtypeStruct((LS, DIM), jnp.float32),
    )(info, q, kv, kv, Wo_bf)

    return out


@jax.jit
def _run(x, Wq, Wk, Wv, Wo, gq, gk):
    x2 = x[0]
    x_ext = jnp.stack(
        [jax.lax.slice(x2, (e0 * BLOCK, 0), (e0 * BLOCK + EXT, DIM))
         for e0 in _E0])
    mesh = Mesh(jax.devices()[:NDEV], ('seq',))
    out = jax.shard_map(
        _local,
        mesh=mesh,
        in_specs=(P('seq', None, None), P(None, None), P(None, None),
                  P(None, None), P(None, None), P(None), P(None)),
        out_specs=P('seq', None),
        check_vma=False,
    )(x_ext, Wq, Wk, Wv, Wo.astype(jnp.bfloat16), gq, gk)
    return out[None]


def kernel(x, Wq, Wk, Wv, Wo, gq, gk):
    return _run(x, Wq, Wk, Wv, Wo, gq, gk)


# speedup vs baseline: 2.7372x; 1.6081x over previous
import jax
import jax.numpy as jnp
from jax.experimental import pallas as pl
from jax.experimental.pallas import tpu as pltpu

S = 2048
DIM = 2048
H = 16
DH = 128
BLOCK = 128
NB = S // BLOCK
WIN = 2
EPS = 1e-6
SCALE = 1.0 / (DH ** 0.5)

STRIP = 256
NSTRIP = S // STRIP
WBLK = STRIP // BLOCK + 2 * WIN
WK = WBLK * BLOCK

NT = 4
TILE_N = DIM // NT


def _qkv_kernel(x_ref, wq_ref, wk_ref, wv_ref, gq_ref, gk_ref, o_ref,
                xs_ref):
    n = pl.program_id(0)

    @pl.when(n == 0)
    def _():
        xs_ref[...] = x_ref[...].astype(jnp.bfloat16)

    def _mm(w_ref):
        w = w_ref[...].astype(jnp.bfloat16)
        return jnp.dot(xs_ref[...], w, preferred_element_type=jnp.float32)

    def _norm(acc, g):
        segs = []
        for j in range(TILE_N // DH):
            seg = acc[:, j * DH:(j + 1) * DH]
            var = jnp.mean(seg * seg, axis=1, keepdims=True)
            segs.append(seg * jax.lax.rsqrt(var + EPS))
        gfull = jnp.concatenate([g] * (TILE_N // DH))
        return (jnp.concatenate(segs, axis=1) * gfull[None, :]).astype(jnp.bfloat16)

    @pl.when(n < NT)
    def _():
        o_ref[...] = _norm(_mm(wq_ref), gq_ref[...])

    @pl.when((n >= NT) & (n < 2 * NT))
    def _():
        o_ref[...] = _norm(_mm(wk_ref), gk_ref[...])

    @pl.when(n >= 2 * NT)
    def _():
        o_ref[...] = _mm(wv_ref).astype(jnp.bfloat16)


def _attn_kernel(q_ref, k_ref, v_ref, wo_ref, o_ref):
    sidx = pl.program_id(0)
    qb0 = sidx * (STRIP // BLOCK)

    NWB = 2 * WIN + 1
    WKH = NWB * BLOCK
    starts, negs = [], []
    for u in range(STRIP // BLOCK):
        sb = jnp.clip(qb0 + u - WIN, 0, NB - NWB)
        c = jax.lax.broadcasted_iota(jnp.int32, (BLOCK, WKH), 1)
        jb = sb + c // BLOCK
        negs.append(jnp.where(jnp.abs(jb - (qb0 + u)) <= WIN,
                              jnp.float32(0), jnp.float32(-1e9)))
        starts.append(sb * BLOCK)

    outs = []
    for h in range(H):
        lo, hi = h * DH, (h + 1) * DH
        halves = []
        for u in range(STRIP // BLOCK):
            qh = q_ref[u * BLOCK:(u + 1) * BLOCK, lo:hi]
            kh = k_ref[pl.ds(starts[u], WKH), lo:hi]
            vh = v_ref[pl.ds(starts[u], WKH), lo:hi]
            s = jax.lax.dot_general(
                qh, kh, (((1,), (1,)), ((), ())),
                preferred_element_type=jnp.float32) * SCALE + negs[u]
            m = jnp.max(s, axis=1, keepdims=True)
            p = jnp.exp(s - m)
            l = jnp.sum(p, axis=1, keepdims=True)
            oh = jnp.dot(p.astype(jnp.bfloat16), vh,
                         preferred_element_type=jnp.float32)
            halves.append((oh / l).astype(jnp.bfloat16))
        outs.append(jnp.concatenate(halves, axis=0))

    a = jnp.concatenate(outs, axis=1)
    o_ref[...] = jnp.dot(a, wo_ref[...], preferred_element_type=jnp.float32)


@jax.jit
def _run(x, Wq, Wk, Wv, Wo, gq, gk):
    qkv = pl.pallas_call(
        _qkv_kernel,
        grid=(3 * NT,),
        in_specs=[
            pl.BlockSpec((S, DIM), lambda n: (0, 0)),
            pl.BlockSpec((DIM, TILE_N),
                         lambda n: (0, jnp.clip(n, 0, NT - 1))),
            pl.BlockSpec((DIM, TILE_N),
                         lambda n: (0, jnp.clip(n - NT, 0, NT - 1))),
            pl.BlockSpec((DIM, TILE_N),
                         lambda n: (0, jnp.clip(n - 2 * NT, 0, NT - 1))),
            pl.BlockSpec((DH,), lambda n: (0,)),
            pl.BlockSpec((DH,), lambda n: (0,)),
        ],
        out_specs=pl.BlockSpec((S, TILE_N), lambda n: (0, n)),
        out_shape=jax.ShapeDtypeStruct((S, 3 * DIM), jnp.bfloat16),
        scratch_shapes=[pltpu.VMEM((S, DIM), jnp.bfloat16)],
    )(x, Wq, Wk, Wv, gq, gk)

    qn = qkv[:, :DIM]
    kn = qkv[:, DIM:2 * DIM]
    vv = qkv[:, 2 * DIM:]

    out = pl.pallas_call(
        _attn_kernel,
        grid=(NSTRIP,),
        in_specs=[
            pl.BlockSpec((STRIP, DIM), lambda s: (s, 0)),
            pl.BlockSpec((S, DIM), lambda s: (0, 0)),
            pl.BlockSpec((S, DIM), lambda s: (0, 0)),
            pl.BlockSpec((DIM, DIM), lambda s: (0, 0)),
        ],
        out_specs=pl.BlockSpec((STRIP, DIM), lambda s: (s, 0)),
        out_shape=jax.ShapeDtypeStruct((S, DIM), jnp.float32),
    )(qn, kn, vv, Wo.astype(jnp.bfloat16))

    return out


def kernel(x, Wq, Wk, Wv, Wo, gq, gk):
    return _run(x[0], Wq, Wk, Wv, Wo, gq, gk)[None]


# speedup vs baseline: 3.2227x; 1.1774x over previous
import jax
import jax.numpy as jnp
from jax.experimental import pallas as pl
from jax.experimental.pallas import tpu as pltpu

S = 2048
DIM = 2048
H = 16
DH = 128
BLOCK = 128
NB = S // BLOCK
WIN = 2
EPS = 1e-6
SCALE = 1.0 / (DH ** 0.5)

STRIP = 256
NSTRIP = S // STRIP
WBLK = STRIP // BLOCK + 2 * WIN
WK = WBLK * BLOCK

NT = 4
TILE_N = DIM // NT


def _qkv_kernel(x_ref, wq_ref, wk_ref, wv_ref, gq_ref, gk_ref, o_ref,
                xs_ref):
    n = pl.program_id(0)

    @pl.when(n == 0)
    def _():
        xs_ref[...] = x_ref[...].astype(jnp.bfloat16)

    def _norm(acc, g):
        segs = []
        for j in range(TILE_N // DH):
            seg = acc[:, j * DH:(j + 1) * DH]
            var = jnp.mean(seg * seg, axis=1, keepdims=True)
            segs.append(seg * jax.lax.rsqrt(var + EPS))
        gfull = jnp.concatenate([g] * (TILE_N // DH))
        return (jnp.concatenate(segs, axis=1) * gfull[None, :]).astype(jnp.bfloat16)

    MSPLIT = 4

    def _mm(w_ref, g):
        w = w_ref[...].astype(jnp.bfloat16)
        parts = []
        for i in range(MSPLIT):
            rows = xs_ref[i * (S // MSPLIT):(i + 1) * (S // MSPLIT), :]
            acc = jnp.dot(rows, w, preferred_element_type=jnp.float32)
            parts.append(_norm(acc, g) if g is not None
                         else acc.astype(jnp.bfloat16))
        return jnp.concatenate(parts, axis=0)

    @pl.when(n < NT)
    def _():
        o_ref[...] = _mm(wq_ref, gq_ref[...])

    @pl.when((n >= NT) & (n < 2 * NT))
    def _():
        o_ref[...] = _mm(wk_ref, gk_ref[...])

    @pl.when(n >= 2 * NT)
    def _():
        o_ref[...] = _mm(wv_ref, None)


def _attn_kernel(q_ref, k_ref, v_ref, wo_ref, o_ref):
    sidx = pl.program_id(0)
    qb0 = sidx * (STRIP // BLOCK)
    start_blk = jnp.clip(qb0 - WIN, 0, NB - WBLK)
    start = start_blk * BLOCK

    r = jax.lax.broadcasted_iota(jnp.int32, (STRIP, WK), 0)
    c = jax.lax.broadcasted_iota(jnp.int32, (STRIP, WK), 1)
    qb = qb0 + r // BLOCK
    jb = start_blk + c // BLOCK
    neg = jnp.where(jnp.abs(jb - qb) <= WIN,
                    jnp.float32(0), jnp.float32(-1e9))

    outs = []
    for h in range(H):
        lo, hi = h * DH, (h + 1) * DH
        qh = q_ref[:, lo:hi]
        kh = k_ref[pl.ds(start, WK), lo:hi]
        vh = v_ref[pl.ds(start, WK), lo:hi]
        s = jax.lax.dot_general(
            qh, kh, (((1,), (1,)), ((), ())),
            preferred_element_type=jnp.float32) * SCALE + neg
        m = jnp.max(s, axis=1, keepdims=True)
        p = jnp.exp(s - m)
        l = jnp.sum(p, axis=1, keepdims=True)
        oh = jnp.dot(p.astype(jnp.bfloat16), vh,
                     preferred_element_type=jnp.float32)
        outs.append((oh / l).astype(jnp.bfloat16))

    a = jnp.concatenate(outs, axis=1)
    o_ref[...] = jnp.dot(a, wo_ref[...], preferred_element_type=jnp.float32)


@jax.jit
def _run(x, Wq, Wk, Wv, Wo, gq, gk):
    qkv = pl.pallas_call(
        _qkv_kernel,
        grid=(3 * NT,),
        in_specs=[
            pl.BlockSpec((S, DIM), lambda n: (0, 0)),
            pl.BlockSpec((DIM, TILE_N),
                         lambda n: (0, jnp.clip(n, 0, NT - 1))),
            pl.BlockSpec((DIM, TILE_N),
                         lambda n: (0, jnp.clip(n - NT, 0, NT - 1))),
            pl.BlockSpec((DIM, TILE_N),
                         lambda n: (0, jnp.clip(n - 2 * NT, 0, NT - 1))),
            pl.BlockSpec((DH,), lambda n: (0,)),
            pl.BlockSpec((DH,), lambda n: (0,)),
        ],
        out_specs=pl.BlockSpec((S, TILE_N), lambda n: (0, n)),
        out_shape=jax.ShapeDtypeStruct((S, 3 * DIM), jnp.bfloat16),
        scratch_shapes=[pltpu.VMEM((S, DIM), jnp.bfloat16)],
    )(x, Wq, Wk, Wv, gq, gk)

    qn = qkv[:, :DIM]
    kn = qkv[:, DIM:2 * DIM]
    vv = qkv[:, 2 * DIM:]

    out = pl.pallas_call(
        _attn_kernel,
        grid=(NSTRIP,),
        in_specs=[
            pl.BlockSpec((STRIP, DIM), lambda s: (s, 0)),
            pl.BlockSpec((S, DIM), lambda s: (0, 0)),
            pl.BlockSpec((S, DIM), lambda s: (0, 0)),
            pl.BlockSpec((DIM, DIM), lambda s: (0, 0)),
        ],
        out_specs=pl.BlockSpec((STRIP, DIM), lambda s: (s, 0)),
        out_shape=jax.ShapeDtypeStruct((S, DIM), jnp.float32),
    )(qn, kn, vv, Wo.astype(jnp.bfloat16))

    return out


def kernel(x, Wq, Wk, Wv, Wo, gq, gk):
    return _run(x[0], Wq, Wk, Wv, Wo, gq, gk)[None]
